# Initial kernel scaffold; baseline (speedup 1.0000x reference)
import functools

import jax
import jax.numpy as jnp
from jax import lax
from jax.experimental import pallas as pl
from jax.experimental.pallas import tpu as pltpu
from jax.experimental.pallas import tpu_sc as plsc

TAU0 = 0.5
NC = 2
NS = 16
LANES = 128


def _rsqrt_approx(d):
  i = plsc.bitcast(d, jnp.int32)
  i = jnp.int32(0x5F3759DF) - lax.shift_right_logical(i, 1)
  y = plsc.bitcast(i, jnp.float32)
  for _ in range(3):
    y = y * (1.5 - 0.5 * d * y * y)
  return y


def _make_deg_kernel(npad, erows):
  mesh = plsc.VectorSubcoreMesh(
      core_axis_name="c", subcore_axis_name="s", num_cores=NC,
      num_subcores=NS)
  sl = npad // NS

  @functools.partial(
      pl.kernel,
      out_type=jax.ShapeDtypeStruct((NC, npad), jnp.float32),
      mesh=mesh,
      scratch_types=[
          pltpu.VMEM((erows, LANES), jnp.int32),
          pltpu.VMEM((LANES,), jnp.float32),
          pltpu.VMEM((sl,), jnp.float32),
          pltpu.VMEM_SHARED((npad,), jnp.float32),
          pltpu.SemaphoreType.DMA,
      ],
  )
  def deg_kernel(dst_hbm, zeros_hbm, ones_hbm, out_hbm,
                 dst_v, ones_v, stage_v, hist_sh, sem):
    c = lax.axis_index("c")
    s = lax.axis_index("s")
    w = c * NS + s

    pltpu.sync_copy(dst_hbm.at[w], dst_v)
    pltpu.sync_copy(ones_hbm, ones_v)

    @pl.when(s == 0)
    def _():
      pltpu.sync_copy(zeros_hbm, hist_sh)

    plsc.subcore_barrier()

    descs = []
    for j in range(erows):
      descs.append(
          pltpu.async_copy(ones_v, hist_sh.at[dst_v.at[j]], sem, add=True))
    for d in descs:
      d.wait()

    plsc.subcore_barrier()
    base = s * sl
    pltpu.sync_copy(hist_sh.at[pl.ds(base, sl)], stage_v)
    pltpu.sync_copy(stage_v, out_hbm.at[c, pl.ds(base, sl)])

  return deg_kernel


def _make_edge_kernel(npad, erows):
  mesh = plsc.VectorSubcoreMesh(
      core_axis_name="c", subcore_axis_name="s", num_cores=NC,
      num_subcores=NS)
  sl = npad // NS

  @functools.partial(
      pl.kernel,
      out_type=jax.ShapeDtypeStruct((NC, npad), jnp.float32),
      mesh=mesh,
      scratch_types=[
          pltpu.VMEM((erows, LANES), jnp.int32),
          pltpu.VMEM((erows, LANES), jnp.int32),
          pltpu.VMEM((erows, LANES), jnp.float32),
          pltpu.VMEM((sl,), jnp.float32),
          pltpu.VMEM((sl,), jnp.float32),
          pltpu.VMEM((sl,), jnp.float32),
          pltpu.VMEM((sl,), jnp.float32),
          pltpu.VMEM_SHARED((npad,), jnp.float32),
          pltpu.VMEM_SHARED((npad,), jnp.float32),
          pltpu.SemaphoreType.DMA,
          pltpu.SemaphoreType.DMA,
      ],
  )
  def edge_kernel(src_hbm, dst_hbm, hist_hbm, h_hbm, zeros_hbm, out_hbm,
                  src_v, dst_v, vals_v, hv, d0, d1, gv,
                  g_sh, acc_sh, sem_g, sem_s):
    c = lax.axis_index("c")
    s = lax.axis_index("s")
    w = c * NS + s
    base = s * sl

    pltpu.sync_copy(src_hbm.at[w], src_v)
    pltpu.sync_copy(dst_hbm.at[w], dst_v)

    @pl.when(s == 0)
    def _():
      pltpu.sync_copy(zeros_hbm, acc_sh)

    pltpu.sync_copy(h_hbm.at[pl.ds(base, sl)], hv)
    pltpu.sync_copy(hist_hbm.at[0, pl.ds(base, sl)], d0)
    pltpu.sync_copy(hist_hbm.at[1, pl.ds(base, sl)], d1)
    for k in range(sl // 16):
      ix = pl.ds(k * 16, 16)
      deg = d0[ix] + d1[ix] + 1.0
      gv[ix] = _rsqrt_approx(deg) * hv[ix]
    pltpu.sync_copy(gv, g_sh.at[pl.ds(base, sl)])

    plsc.subcore_barrier()

    gdescs = []
    for j in range(erows):
      gdescs.append(
          pltpu.async_copy(g_sh.at[src_v.at[j]], vals_v.at[j], sem_g))
    for d in gdescs:
      d.wait()
    sdescs = []
    for j in range(erows):
      sdescs.append(
          pltpu.async_copy(vals_v.at[j], acc_sh.at[dst_v.at[j]], sem_s,
                           add=True))
    for d in sdescs:
      d.wait()

    plsc.subcore_barrier()
    pltpu.sync_copy(acc_sh.at[pl.ds(base, sl)], gv)
    pltpu.sync_copy(gv, out_hbm.at[c, pl.ds(base, sl)])

  return edge_kernel


def _matvec(x, w_row):
  n, d = x.shape
  blocks = 8
  rb = n // blocks

  def body(x_ref, w_ref, o_ref):
    o_ref[0, :] = jnp.sum(x_ref[...] * w_ref[...], axis=1)

  return pl.pallas_call(
      body,
      grid=(blocks,),
      in_specs=[
          pl.BlockSpec((rb, d), lambda i: (i, 0)),
          pl.BlockSpec((1, d), lambda i: (0, 0)),
      ],
      out_specs=pl.BlockSpec((1, rb), lambda i: (i, 0)),
      out_shape=jax.ShapeDtypeStruct((blocks, rb), jnp.float32),
  )(x, w_row)


def _epilogue(hist, acc, h_row):
  npad = h_row.shape[1]

  def body(hist_ref, acc_ref, h_ref, o_ref):
    deg = hist_ref[0:1, :] + hist_ref[1:2, :] + 1.0
    dinv = lax.rsqrt(deg)
    a = acc_ref[0:1, :] + acc_ref[1:2, :]
    out = dinv * a + dinv * dinv * h_ref[...]
    sp = jnp.maximum(out, 0.0) + jnp.log1p(jnp.exp(-jnp.abs(out))) + TAU0
    o_ref[...] = 1.0 / sp

  return pl.pallas_call(
      body,
      out_shape=jax.ShapeDtypeStruct((1, npad), jnp.float32),
  )(hist, acc, h_row)


def kernel(x, edge_index, edge_attr, W):
  n = x.shape[0]
  e = edge_index.shape[1]
  npad = (n // 256 + 1) * 256
  ept = -(-e // (NC * NS * LANES)) * LANES
  erows = ept // LANES
  epad = NC * NS * ept

  npads = npad - n
  pad_idx = n + (jnp.arange(epad - e, dtype=jnp.int32) % npads)
  src = jnp.concatenate([edge_index[0], pad_idx]).reshape(NC * NS, erows,
                                                          LANES)
  dst = jnp.concatenate([edge_index[1], pad_idx]).reshape(NC * NS, erows,
                                                          LANES)

  zeros = jnp.zeros((npad,), jnp.float32)
  ones = jnp.ones((LANES,), jnp.float32)

  hist = _make_deg_kernel(npad, erows)(dst, zeros, ones)

  h = _matvec(x, W.reshape(1, -1)).reshape(1, n)
  h_row = jnp.concatenate([h, jnp.zeros((1, npad - n), jnp.float32)], axis=1)

  acc = _make_edge_kernel(npad, erows)(src, dst, hist, h_row.reshape(npad),
                                       zeros)

  temp = _epilogue(hist, acc, h_row)
  return temp[0, :n, None]


# speedup vs baseline: 62.1691x; 62.1691x over previous
import functools

import jax
import jax.numpy as jnp
from jax import lax
from jax.experimental import pallas as pl
from jax.experimental.pallas import tpu as pltpu
from jax.experimental.pallas import tpu_sc as plsc

TAU0 = 0.5
NC = 2
NS = 16
LANES = 128


def _make_deg_kernel(npad, erows):
  mesh = plsc.VectorSubcoreMesh(
      core_axis_name="c", subcore_axis_name="s", num_cores=NC,
      num_subcores=NS)
  sl = npad // NS

  @functools.partial(
      pl.kernel,
      out_type=jax.ShapeDtypeStruct((NC, npad), jnp.float32),
      mesh=mesh,
      scratch_types=[
          pltpu.VMEM((erows, LANES), jnp.int32),
          pltpu.VMEM((LANES,), jnp.float32),
          pltpu.VMEM((sl,), jnp.float32),
          pltpu.VMEM_SHARED((npad,), jnp.float32),
          pltpu.SemaphoreType.DMA,
      ],
  )
  def deg_kernel(dst_hbm, zeros_hbm, ones_hbm, out_hbm,
                 dst_v, ones_v, stage_v, hist_sh, sem):
    c = lax.axis_index("c")
    s = lax.axis_index("s")
    w = c * NS + s

    pltpu.sync_copy(dst_hbm.at[w], dst_v)
    pltpu.sync_copy(ones_hbm, ones_v)

    @pl.when(s == 0)
    def _():
      pltpu.sync_copy(zeros_hbm, hist_sh)

    plsc.subcore_barrier()

    descs = []
    for j in range(erows):
      descs.append(
          pltpu.async_copy(ones_v, hist_sh.at[dst_v.at[j]], sem, add=True))
    for d in descs:
      d.wait()

    plsc.subcore_barrier()
    base = s * sl
    pltpu.sync_copy(hist_sh.at[pl.ds(base, sl)], stage_v)
    pltpu.sync_copy(stage_v, out_hbm.at[c, pl.ds(base, sl)])

  return deg_kernel


def _make_edge_kernel(npad, erows):
  mesh = plsc.VectorSubcoreMesh(
      core_axis_name="c", subcore_axis_name="s", num_cores=NC,
      num_subcores=NS)
  sl = npad // NS

  @functools.partial(
      pl.kernel,
      out_type=jax.ShapeDtypeStruct((NC, npad), jnp.float32),
      mesh=mesh,
      scratch_types=[
          pltpu.VMEM((erows, LANES), jnp.int32),
          pltpu.VMEM((erows, LANES), jnp.int32),
          pltpu.VMEM((erows, LANES), jnp.float32),
          pltpu.VMEM((sl,), jnp.float32),
          pltpu.VMEM_SHARED((npad,), jnp.float32),
          pltpu.VMEM_SHARED((npad,), jnp.float32),
          pltpu.SemaphoreType.DMA,
          pltpu.SemaphoreType.DMA,
      ],
  )
  def edge_kernel(src_hbm, dst_hbm, g_hbm, zeros_hbm, out_hbm,
                  src_v, dst_v, vals_v, gv,
                  g_sh, acc_sh, sem_g, sem_s):
    c = lax.axis_index("c")
    s = lax.axis_index("s")
    w = c * NS + s
    base = s * sl

    pltpu.sync_copy(src_hbm.at[w], src_v)
    pltpu.sync_copy(dst_hbm.at[w], dst_v)

    @pl.when(s == 0)
    def _():
      pltpu.sync_copy(zeros_hbm, acc_sh)

    pltpu.sync_copy(g_hbm.at[pl.ds(base, sl)], gv)
    pltpu.sync_copy(gv, g_sh.at[pl.ds(base, sl)])

    plsc.subcore_barrier()

    gdescs = []
    for j in range(erows):
      gdescs.append(
          pltpu.async_copy(g_sh.at[src_v.at[j]], vals_v.at[j], sem_g))
    for d in gdescs:
      d.wait()
    sdescs = []
    for j in range(erows):
      sdescs.append(
          pltpu.async_copy(vals_v.at[j], acc_sh.at[dst_v.at[j]], sem_s,
                           add=True))
    for d in sdescs:
      d.wait()

    plsc.subcore_barrier()
    pltpu.sync_copy(acc_sh.at[pl.ds(base, sl)], gv)
    pltpu.sync_copy(gv, out_hbm.at[c, pl.ds(base, sl)])

  return edge_kernel


def _matvec_g(x, w_row, hist):
  n, d = x.shape
  blocks = 10
  rb = n // blocks

  def body(x_ref, w_ref, hist_ref, h_ref, g_ref):
    i = pl.program_id(0)
    s = jnp.sum(x_ref[...] * w_ref[...], axis=1)
    deg = hist_ref[0, 0, :] + hist_ref[0, 1, :] + 1.0
    g = lax.rsqrt(deg) * s
    h_ref[pl.ds(i, 1), :] = s.reshape(1, rb)
    g_ref[pl.ds(i, 1), :] = g.reshape(1, rb)

  return pl.pallas_call(
      body,
      grid=(blocks,),
      in_specs=[
          pl.BlockSpec((rb, d), lambda i: (i, 0)),
          pl.BlockSpec((1, d), lambda i: (0, 0)),
          pl.BlockSpec((1, 2, rb), lambda i: (i, 0, 0)),
      ],
      out_specs=[
          pl.BlockSpec((blocks, rb), lambda i: (0, 0)),
          pl.BlockSpec((blocks, rb), lambda i: (0, 0)),
      ],
      out_shape=[
          jax.ShapeDtypeStruct((blocks, rb), jnp.float32),
          jax.ShapeDtypeStruct((blocks, rb), jnp.float32),
      ],
  )(x, w_row, hist[:, :n].reshape(2, blocks, rb).transpose(1, 0, 2))


def _epilogue(hist, acc, h_row):
  npad = h_row.shape[1]

  def body(hist_ref, acc_ref, h_ref, o_ref):
    deg = hist_ref[0:1, :] + hist_ref[1:2, :] + 1.0
    dinv = lax.rsqrt(deg)
    a = acc_ref[0:1, :] + acc_ref[1:2, :]
    out = dinv * a + dinv * dinv * h_ref[...]
    sp = jnp.maximum(out, 0.0) + jnp.log1p(jnp.exp(-jnp.abs(out))) + TAU0
    o_ref[...] = 1.0 / sp

  return pl.pallas_call(
      body,
      out_shape=jax.ShapeDtypeStruct((1, npad), jnp.float32),
  )(hist, acc, h_row)


def kernel(x, edge_index, edge_attr, W):
  n = x.shape[0]
  e = edge_index.shape[1]
  npad = (n // 256 + 1) * 256
  ept = -(-e // (NC * NS * LANES)) * LANES
  erows = ept // LANES
  epad = NC * NS * ept

  npads = npad - n
  pad_idx = n + (jnp.arange(epad - e, dtype=jnp.int32) % npads)
  src = jnp.concatenate([edge_index[0], pad_idx]).reshape(NC * NS, erows,
                                                          LANES)
  dst = jnp.concatenate([edge_index[1], pad_idx]).reshape(NC * NS, erows,
                                                          LANES)

  zeros = jnp.zeros((npad,), jnp.float32)
  ones = jnp.ones((LANES,), jnp.float32)

  hist = _make_deg_kernel(npad, erows)(dst, zeros, ones)

  h, g = _matvec_g(x, W.reshape(1, -1), hist)
  pad = jnp.zeros((1, npad - n), jnp.float32)
  h_row = jnp.concatenate([h.reshape(1, n), pad], axis=1)
  g_pad = jnp.concatenate([g.reshape(n), pad.reshape(npad - n)])

  acc = _make_edge_kernel(npad, erows)(src, dst, g_pad, zeros)

  temp = _epilogue(hist, acc, h_row)
  return temp[0, :n, None]


# speedup vs baseline: 63.5085x; 1.0215x over previous
import functools

import jax
import jax.numpy as jnp
from jax import lax
from jax.experimental import pallas as pl
from jax.experimental.pallas import tpu as pltpu
from jax.experimental.pallas import tpu_sc as plsc

TAU0 = 0.5
NC = 2
NS = 16
LANES = 128


def _make_deg_kernel(npad, erows):
  mesh = plsc.VectorSubcoreMesh(
      core_axis_name="c", subcore_axis_name="s", num_cores=NC,
      num_subcores=NS)
  sl = npad // NS

  @functools.partial(
      pl.kernel,
      out_type=jax.ShapeDtypeStruct((NC, npad), jnp.float32),
      mesh=mesh,
      scratch_types=[
          pltpu.VMEM((erows, LANES), jnp.int32),
          pltpu.VMEM((LANES,), jnp.float32),
          pltpu.VMEM((sl,), jnp.float32),
          pltpu.VMEM_SHARED((npad,), jnp.float32),
          pltpu.SemaphoreType.DMA,
      ],
  )
  def deg_kernel(dst_hbm, zeros_hbm, ones_hbm, out_hbm,
                 dst_v, ones_v, stage_v, hist_sh, sem):
    c = lax.axis_index("c")
    s = lax.axis_index("s")
    w = c * NS + s

    pltpu.sync_copy(dst_hbm.at[w], dst_v)
    pltpu.sync_copy(ones_hbm, ones_v)

    @pl.when(s == 0)
    def _():
      pltpu.sync_copy(zeros_hbm, hist_sh)

    plsc.subcore_barrier()

    descs = []
    for j in range(erows):
      descs.append(
          pltpu.async_copy(ones_v, hist_sh.at[dst_v.at[j]], sem, add=True))
    for d in descs:
      d.wait()

    plsc.subcore_barrier()
    base = s * sl
    pltpu.sync_copy(hist_sh.at[pl.ds(base, sl)], stage_v)
    pltpu.sync_copy(stage_v, out_hbm.at[c, pl.ds(base, sl)])

  return deg_kernel


def _make_edge_kernel(npad, erows):
  mesh = plsc.VectorSubcoreMesh(
      core_axis_name="c", subcore_axis_name="s", num_cores=NC,
      num_subcores=NS)
  sl = npad // NS

  @functools.partial(
      pl.kernel,
      out_type=jax.ShapeDtypeStruct((NC, npad), jnp.float32),
      mesh=mesh,
      scratch_types=[
          pltpu.VMEM((erows, LANES), jnp.int32),
          pltpu.VMEM((erows, LANES), jnp.int32),
          pltpu.VMEM((erows, LANES), jnp.float32),
          pltpu.VMEM((sl,), jnp.float32),
          pltpu.VMEM_SHARED((npad,), jnp.float32),
          pltpu.VMEM_SHARED((npad,), jnp.float32),
          pltpu.SemaphoreType.DMA,
          pltpu.SemaphoreType.DMA,
      ],
  )
  def edge_kernel(src_hbm, dst_hbm, g_hbm, zeros_hbm, out_hbm,
                  src_v, dst_v, vals_v, gv,
                  g_sh, acc_sh, sem_g, sem_s):
    c = lax.axis_index("c")
    s = lax.axis_index("s")
    w = c * NS + s
    base = s * sl

    pltpu.sync_copy(src_hbm.at[w], src_v)
    pltpu.sync_copy(dst_hbm.at[w], dst_v)

    @pl.when(s == 0)
    def _():
      pltpu.sync_copy(zeros_hbm, acc_sh)

    pltpu.sync_copy(g_hbm.at[pl.ds(base, sl)], gv)
    pltpu.sync_copy(gv, g_sh.at[pl.ds(base, sl)])

    plsc.subcore_barrier()

    gdescs = []
    for j in range(erows):
      gdescs.append(
          pltpu.async_copy(g_sh.at[src_v.at[j]], vals_v.at[j], sem_g))
    for d in gdescs:
      d.wait()
    sdescs = []
    for j in range(erows):
      sdescs.append(
          pltpu.async_copy(vals_v.at[j], acc_sh.at[dst_v.at[j]], sem_s,
                           add=True))
    for d in sdescs:
      d.wait()

    plsc.subcore_barrier()
    pltpu.sync_copy(acc_sh.at[pl.ds(base, sl)], gv)
    pltpu.sync_copy(gv, out_hbm.at[c, pl.ds(base, sl)])

  return edge_kernel


def _matvec_g(x, w_row, hist):
  n, d = x.shape
  blocks = 10
  rb = n // blocks

  def body(x_ref, w_ref, hist_ref, h_ref, g_ref):
    i = pl.program_id(0)
    s = lax.dot_general(w_ref[...], x_ref[...], (((1,), (1,)), ((), ())),
                        preferred_element_type=jnp.float32)
    deg = hist_ref[0, 0:1, :] + hist_ref[0, 1:2, :] + 1.0
    g = lax.rsqrt(deg) * s
    h_ref[pl.ds(i, 1), :] = s
    g_ref[pl.ds(i, 1), :] = g

  return pl.pallas_call(
      body,
      grid=(blocks,),
      in_specs=[
          pl.BlockSpec((rb, d), lambda i: (i, 0)),
          pl.BlockSpec((1, d), lambda i: (0, 0)),
          pl.BlockSpec((1, 2, rb), lambda i: (i, 0, 0)),
      ],
      out_specs=[
          pl.BlockSpec((blocks, rb), lambda i: (0, 0)),
          pl.BlockSpec((blocks, rb), lambda i: (0, 0)),
      ],
      out_shape=[
          jax.ShapeDtypeStruct((blocks, rb), jnp.float32),
          jax.ShapeDtypeStruct((blocks, rb), jnp.float32),
      ],
  )(x, w_row, hist[:, :n].reshape(2, blocks, rb).transpose(1, 0, 2))


def _epilogue(hist, acc, h_row):
  npad = h_row.shape[1]

  def body(hist_ref, acc_ref, h_ref, o_ref):
    deg = hist_ref[0:1, :] + hist_ref[1:2, :] + 1.0
    dinv = lax.rsqrt(deg)
    a = acc_ref[0:1, :] + acc_ref[1:2, :]
    out = dinv * a + dinv * dinv * h_ref[...]
    sp = jnp.maximum(out, 0.0) + jnp.log1p(jnp.exp(-jnp.abs(out))) + TAU0
    o_ref[...] = 1.0 / sp

  return pl.pallas_call(
      body,
      out_shape=jax.ShapeDtypeStruct((1, npad), jnp.float32),
  )(hist, acc, h_row)


def kernel(x, edge_index, edge_attr, W):
  n = x.shape[0]
  e = edge_index.shape[1]
  npad = (n // 256 + 1) * 256
  ept = -(-e // (NC * NS * LANES)) * LANES
  erows = ept // LANES
  epad = NC * NS * ept

  npads = npad - n
  pad_idx = n + (jnp.arange(epad - e, dtype=jnp.int32) % npads)
  flat = edge_index.reshape(2 * e)
  src = jnp.concatenate([flat[:e], pad_idx]).reshape(NC * NS, erows, LANES)
  dst = jnp.concatenate([flat[e:], pad_idx]).reshape(NC * NS, erows, LANES)

  zeros = jnp.zeros((npad,), jnp.float32)
  ones = jnp.ones((LANES,), jnp.float32)

  hist = _make_deg_kernel(npad, erows)(dst, zeros, ones)

  h, g = _matvec_g(x, W.reshape(1, -1), hist)
  pad = jnp.zeros((1, npad - n), jnp.float32)
  h_row = jnp.concatenate([h.reshape(1, n), pad], axis=1)
  g_pad = jnp.concatenate([g.reshape(n), pad.reshape(npad - n)])

  acc = _make_edge_kernel(npad, erows)(src, dst, g_pad, zeros)

  temp = _epilogue(hist, acc, h_row)
  return temp[0, :n, None]
